# baseline (speedup 1.0000x reference)
import functools

import jax
import jax.numpy as jnp
from jax import lax
from jax.experimental import pallas as pl
from jax.experimental.pallas import tpu as pltpu
from jax.experimental.pallas import tpu_sc as plsc

_C = 64
_SLOTS = 72
_NW = 32
_WIN = 32
_LANES = 16


def _sc_span_sums(hidden_flat, idx, n_spans, d):
    spw = n_spans // _NW
    mesh = plsc.VectorSubcoreMesh(core_axis_name="c", subcore_axis_name="s")

    @functools.partial(
        pl.kernel,
        out_type=jax.ShapeDtypeStruct((_NW, n_spans // _NW, d), jnp.float32),
        mesh=mesh,
        scratch_types=[
            pltpu.VMEM((spw, _WIN), jnp.int32),
            pltpu.VMEM((_WIN, d), jnp.float32),
            pltpu.VMEM((spw, d), jnp.float32),
            pltpu.SemaphoreType.DMA,
        ],
    )
    def body(hid_hbm, idx_hbm, out_hbm, idx_v, rows_v, sums_v, sem):
        wid = lax.axis_index("c") * 16 + lax.axis_index("s")
        pltpu.sync_copy(idx_hbm.at[wid], idx_v)

        def span_body(t, carry):
            pltpu.async_copy(hid_hbm.at[idx_v.at[t]], rows_v, sem).wait()

            def chunk_body(k, carry2):
                def row_body(j, acc):
                    return acc + rows_v[j, pl.ds(k * _LANES, _LANES)]

                acc = lax.fori_loop(
                    0, _WIN, row_body, jnp.zeros((_LANES,), jnp.float32)
                )
                sums_v[t, pl.ds(k * _LANES, _LANES)] = acc
                return carry2

            lax.fori_loop(0, d // _LANES, chunk_body, 0)
            return carry

        lax.fori_loop(0, spw, span_body, 0)
        pltpu.sync_copy(sums_v, out_hbm.at[wid])

    return body(hidden_flat, idx)


def _tc_finish(sums, inv_len, pad_scale, h00, n_valid):
    b, slots, d = sums.shape

    def body(sums_ref, inv_ref, pad_ref, h00_ref, nv_ref, out_ref):
        means = (
            sums_ref[:] * inv_ref[:][:, :, None]
            - pad_ref[:][:, :, None] * h00_ref[:][None, :, :]
        )
        cand = means[:, :_C, :]
        num = means[:, _C:_C + 1, :]
        dots = jnp.sum(cand * num, axis=-1)
        cid = lax.broadcasted_iota(jnp.int32, (b, _C), 1)
        out_ref[:] = jnp.where(cid < nv_ref[:], dots, 0.0)

    return pl.pallas_call(
        body,
        out_shape=jax.ShapeDtypeStruct((b, _C), jnp.float32),
    )(sums, inv_len, pad_scale, h00, n_valid)


def kernel(last_hidden, cand_starts, cand_lens, num_starts, num_lens, n_valid):
    B, S, D = last_hidden.shape
    n_spans = B * _SLOTS

    cand_starts = cand_starts.astype(jnp.int32)
    cand_lens = cand_lens.astype(jnp.int32)
    num_starts = num_starts.astype(jnp.int32)
    num_lens = num_lens.astype(jnp.int32)

    pad = _SLOTS - _C - 1
    starts = jnp.concatenate(
        [cand_starts, num_starts[:, None], jnp.zeros((B, pad), jnp.int32)], axis=1
    )
    lens = jnp.concatenate(
        [cand_lens, num_lens[:, None], jnp.ones((B, pad), jnp.int32)], axis=1
    )
    lens = jnp.maximum(lens, 1)
    starts = jnp.clip(starts, 0, S - 1)
    ends = jnp.clip(starts + lens, 1, S)
    eff = ends - starts

    base = starts + jnp.arange(B, dtype=jnp.int32)[:, None] * S
    j = jnp.arange(_WIN, dtype=jnp.int32)
    idx = jnp.where(
        j[None, None, :] < eff[:, :, None], base[:, :, None] + j[None, None, :], 0
    ).astype(jnp.int32)
    idx = idx.reshape(_NW, n_spans // _NW, _WIN)

    hidden_flat = last_hidden.reshape(B * S, D)
    sums = _sc_span_sums(hidden_flat, idx, n_spans, D).reshape(B, _SLOTS, D)


    efff = eff.astype(jnp.float32)
    inv_len = 1.0 / efff
    pad_scale = (_WIN - efff) / efff
    h00 = hidden_flat[0:1]

    return _tc_finish(sums, inv_len, pad_scale, h00, n_valid.astype(jnp.int32)[:, None])


# speedup vs baseline: 1.1054x; 1.1054x over previous
import functools

import jax
import jax.numpy as jnp
from jax import lax
from jax.experimental import pallas as pl
from jax.experimental.pallas import tpu as pltpu
from jax.experimental.pallas import tpu_sc as plsc

_C = 64
_SLOTS = 68
_NW = 32
_WIN = 32
_LANES = 16


def _sc_span_sums(hidden_flat, idx, n_spans, d):
    spw = n_spans // _NW
    mesh = plsc.VectorSubcoreMesh(core_axis_name="c", subcore_axis_name="s")

    @functools.partial(
        pl.kernel,
        out_type=jax.ShapeDtypeStruct((_NW, spw, d), jnp.float32),
        mesh=mesh,
        scratch_types=[
            pltpu.VMEM((spw, _WIN), jnp.int32),
            pltpu.VMEM((_WIN, d), jnp.float32),
            pltpu.VMEM((_WIN, d), jnp.float32),
            pltpu.VMEM((spw, d), jnp.float32),
            pltpu.SemaphoreType.DMA,
            pltpu.SemaphoreType.DMA,
        ],
    )
    def body(hid_hbm, idx_hbm, out_hbm, idx_v, rows0_v, rows1_v, sums_v, sem0, sem1):
        wid = lax.axis_index("c") * 16 + lax.axis_index("s")
        pltpu.sync_copy(idx_hbm.at[wid], idx_v)
        rows = (rows0_v, rows1_v)
        sems = (sem0, sem1)

        pltpu.async_copy(hid_hbm.at[idx_v.at[0]], rows0_v, sem0)
        pltpu.async_copy(hid_hbm.at[idx_v.at[1]], rows1_v, sem1)

        def pair_body(g, carry):
            for b in range(2):
                t = g * 2 + b
                pltpu.make_async_copy(
                    hid_hbm.at[idx_v.at[t]], rows[b], sems[b]
                ).wait()

                def chunk_body(k, c2, _b=b):
                    buf = rows[_b]
                    accs = [jnp.zeros((_LANES,), jnp.float32) for _ in range(4)]
                    for j in range(_WIN):
                        accs[j % 4] = accs[j % 4] + buf[j, pl.ds(k * _LANES, _LANES)]
                    sums_v[t, pl.ds(k * _LANES, _LANES)] = (
                        (accs[0] + accs[1]) + (accs[2] + accs[3])
                    )
                    return c2

                lax.fori_loop(0, d // _LANES, chunk_body, 0)

                @pl.when(t + 2 < spw)
                def _refill(_b=b):
                    pltpu.async_copy(
                        hid_hbm.at[idx_v.at[t + 2]], rows[_b], sems[_b]
                    )

            return carry

        lax.fori_loop(0, spw // 2, pair_body, 0)
        pltpu.sync_copy(sums_v, out_hbm.at[wid])

    return body(hidden_flat, idx)


def _tc_finish(sums, inv_len, pad_scale, h00, n_valid):
    b, slots, d = sums.shape

    def body(sums_ref, inv_ref, pad_ref, h00_ref, nv_ref, out_ref):
        means = (
            sums_ref[:] * inv_ref[:][:, :, None]
            - pad_ref[:][:, :, None] * h00_ref[:][None, :, :]
        )
        cand = means[:, :_C, :]
        num = means[:, _C:_C + 1, :]
        dots = jnp.sum(cand * num, axis=-1)
        cid = lax.broadcasted_iota(jnp.int32, (b, _C), 1)
        out_ref[:] = jnp.where(cid < nv_ref[:], dots, 0.0)

    return pl.pallas_call(
        body,
        out_shape=jax.ShapeDtypeStruct((b, _C), jnp.float32),
    )(sums, inv_len, pad_scale, h00, n_valid)


def kernel(last_hidden, cand_starts, cand_lens, num_starts, num_lens, n_valid):
    B, S, D = last_hidden.shape
    n_spans = B * _SLOTS

    cand_starts = cand_starts.astype(jnp.int32)
    cand_lens = cand_lens.astype(jnp.int32)
    num_starts = num_starts.astype(jnp.int32)
    num_lens = num_lens.astype(jnp.int32)

    pad = _SLOTS - _C - 1
    starts = jnp.concatenate(
        [cand_starts, num_starts[:, None], jnp.zeros((B, pad), jnp.int32)], axis=1
    )
    lens = jnp.concatenate(
        [cand_lens, num_lens[:, None], jnp.ones((B, pad), jnp.int32)], axis=1
    )
    lens = jnp.maximum(lens, 1)
    starts = jnp.clip(starts, 0, S - 1)
    ends = jnp.clip(starts + lens, 1, S)
    eff = ends - starts

    base = starts + jnp.arange(B, dtype=jnp.int32)[:, None] * S
    j = jnp.arange(_WIN, dtype=jnp.int32)
    idx = jnp.where(
        j[None, None, :] < eff[:, :, None], base[:, :, None] + j[None, None, :], 0
    ).astype(jnp.int32)
    idx = idx.reshape(_NW, n_spans // _NW, _WIN)

    hidden_flat = last_hidden.reshape(B * S, D)
    sums = _sc_span_sums(hidden_flat, idx, n_spans, D).reshape(B, _SLOTS, D)

    efff = eff.astype(jnp.float32)
    inv_len = 1.0 / efff
    pad_scale = (_WIN - efff) / efff
    h00 = hidden_flat[0:1]

    return _tc_finish(sums, inv_len, pad_scale, h00, n_valid.astype(jnp.int32)[:, None])
